# baseline (speedup 1.0000x reference)
import functools

import jax
import jax.numpy as jnp
from jax import lax
from jax.experimental import pallas as pl
from jax.experimental.pallas import tpu as pltpu
from jax.experimental.pallas import tpu_sc as plsc

N_NODES = 10000
H = 128
L = 16
NC = 2
NS = 16
NW = NC * NS
GB = 128
CH = 256
ROWS_PER_SUB = N_NODES // NS


def _bcast_lane(v, lane):
    return lax.gather(
        v, jnp.full((L, 1), lane, jnp.int32),
        dimension_numbers=lax.GatherDimensionNumbers(
            offset_dims=(), collapsed_slice_dims=(0,), start_index_map=(0,)),
        slice_sizes=(1,), mode=lax.GatherScatterMode.PROMISE_IN_BOUNDS)


def _edge_scatter(x, rbf_p, idx_g, wt8):
    E = x.shape[0]
    assert E % CH == 0
    nchunk_total = E // CH
    nchunk_base = nchunk_total // NW
    nchunk_rem = nchunk_total % NW
    g_per_chunk = CH // GB

    mesh = plsc.VectorSubcoreMesh(core_axis_name="c", subcore_axis_name="s")

    @functools.partial(
        pl.kernel,
        out_type=jax.ShapeDtypeStruct((NC, N_NODES, H), jnp.float32),
        mesh=mesh,
        scratch_types=[
            pltpu.VMEM((CH, H), jnp.float32),
            pltpu.VMEM((CH // 2, L), jnp.float32),
            pltpu.VMEM((g_per_chunk, GB), jnp.int32),
            pltpu.VMEM((8, H), jnp.float32),
            pltpu.VMEM_SHARED((N_NODES, H), jnp.float32),
        ],
    )
    def body(x_hbm, rbf_hbm, idx_hbm, wt_hbm, out_hbm, xb, rb, ib, wt, acc):
        cid = lax.axis_index("c")
        sid = lax.axis_index("s")
        wid = sid * NC + cid

        pltpu.sync_copy(wt_hbm, wt)

        def zrow(j, _):
            for k in range(H // L):
                xb[j, pl.ds(k * L, L)] = jnp.zeros((L,), jnp.float32)
            return 0
        lax.fori_loop(0, GB, zrow, 0)

        @pl.when(sid < NS - 1)
        def _():
            for j in range(6):
                pltpu.sync_copy(xb.at[pl.ds(0, 104)],
                                acc.at[pl.ds(sid * 624 + j * 104, 104)])

        @pl.when(sid == NS - 1)
        def _():
            for j in range(5):
                pltpu.sync_copy(xb.at[pl.ds(0, 128)],
                                acc.at[pl.ds(9360 + j * 128, 128)])

        plsc.subcore_barrier()

        nchunk = nchunk_base + jnp.where(wid < nchunk_rem, 1, 0)

        def chunk_body(c, _):
            t = wid + c * NW
            pltpu.sync_copy(x_hbm.at[pl.ds(t * CH, CH)], xb)
            pltpu.sync_copy(rbf_hbm.at[pl.ds(t * (CH // 2), CH // 2)], rb)
            for g in range(g_per_chunk):
                pltpu.sync_copy(idx_hbm.at[pl.ds(t * CH + g * GB, GB)],
                                ib.at[g])

            def pair_body(j, _):
                rv = rb[j]
                for half in range(2):
                    e = 2 * j + half
                    br = [_bcast_lane(rv, 8 * half + r) for r in range(6)]
                    for k in range(H // L):
                        w = br[0] * wt[0, pl.ds(k * L, L)]
                        for r in range(1, 6):
                            w = w + br[r] * wt[r, pl.ds(k * L, L)]
                        xb[e, pl.ds(k * L, L)] = xb[e, pl.ds(k * L, L)] * w
                return 0
            lax.fori_loop(0, CH // 2, pair_body, 0)

            for g in range(g_per_chunk):
                pltpu.sync_copy(xb.at[pl.ds(g * GB, GB)],
                                acc.at[ib.at[g]], add=True)
            return 0
        lax.fori_loop(0, nchunk, chunk_body, 0)

        plsc.subcore_barrier()

        @pl.when(sid < NS - 1)
        def _():
            pltpu.sync_copy(acc.at[pl.ds(sid * 624, 624)],
                            out_hbm.at[cid, pl.ds(sid * 624, 624)])

        @pl.when(sid == NS - 1)
        def _():
            pltpu.sync_copy(acc.at[pl.ds(9360, 640)],
                            out_hbm.at[cid, pl.ds(9360, 640)])

    return body(x, rbf_p, idx_g, wt8)


def _mlp(partials, W0, b0, W1, b1, W2, b2, W_out):
    BN = 2000
    grid = (N_NODES // BN,)

    def body(p_ref, w0, b0r, w1, b1r, w2, b2r, wo, o_ref):
        dn = (((1,), (1,)), ((), ()))
        h = p_ref[0] + p_ref[1]
        h = jnp.maximum(
            lax.dot_general(h, w0[...], dn, preferred_element_type=jnp.float32)
            + b0r[...], 0.0)
        h = jnp.maximum(
            lax.dot_general(h, w1[...], dn, preferred_element_type=jnp.float32)
            + b1r[...], 0.0)
        h = jnp.maximum(
            lax.dot_general(h, w2[...], dn, preferred_element_type=jnp.float32)
            + b2r[...], 0.0)
        o_ref[...] = lax.dot_general(h, wo[...], dn,
                                     preferred_element_type=jnp.float32)

    wspec = pl.BlockSpec((H, H), lambda i: (0, 0))
    bspec = pl.BlockSpec((1, H), lambda i: (0, 0))
    return pl.pallas_call(
        body,
        grid=grid,
        in_specs=[
            pl.BlockSpec((NC, BN, H), lambda i: (0, i, 0)),
            wspec, bspec, wspec, bspec, wspec, bspec, wspec,
        ],
        out_specs=pl.BlockSpec((BN, H), lambda i: (i, 0)),
        out_shape=jax.ShapeDtypeStruct((N_NODES, H), jnp.float32),
    )(partials, W0, b0, W1, b1, W2, b2, W_out)


def kernel(x, rbf, i, num_nodes, W_rbf, W0, b0, W1, b1, W2, b2, W_out):
    E = x.shape[0]
    idx = jnp.minimum(i, N_NODES - 1).astype(jnp.int32)
    rbf8 = jnp.concatenate(
        [rbf.astype(jnp.float32), jnp.zeros((E, 2), jnp.float32)], axis=1)
    rbf_p = rbf8.reshape(E // 2, 16)
    idx_g = idx
    wt8 = jnp.zeros((8, H), jnp.float32).at[:6, :].set(W_rbf.T)

    partials = _edge_scatter(x, rbf_p, idx_g, wt8)
    return _mlp(partials, W0, b0.reshape(1, H), W1, b1.reshape(1, H),
                W2, b2.reshape(1, H), W_out)


# speedup vs baseline: 2.3626x; 2.3626x over previous
import functools

import jax
import jax.numpy as jnp
from jax import lax
from jax.experimental import pallas as pl
from jax.experimental.pallas import tpu as pltpu
from jax.experimental.pallas import tpu_sc as plsc

N_NODES = 10000
H = 128
L = 16
NC = 2
NS = 16
NW = NC * NS
CH = 64


def _bcast_lane(v, lane):
    return lax.gather(
        v, jnp.full((L, 1), lane, jnp.int32),
        dimension_numbers=lax.GatherDimensionNumbers(
            offset_dims=(), collapsed_slice_dims=(0,), start_index_map=(0,)),
        slice_sizes=(1,), mode=lax.GatherScatterMode.PROMISE_IN_BOUNDS)


def _edge_scatter(x, rbf_p, idx, wt8):
    E = x.shape[0]
    assert E % CH == 0
    nchunk = E // CH
    ncmax = (nchunk + NW - 1) // NW

    mesh = plsc.VectorSubcoreMesh(core_axis_name="c", subcore_axis_name="s")

    @functools.partial(
        pl.kernel,
        out_type=jax.ShapeDtypeStruct((NC, N_NODES, H), jnp.float32),
        mesh=mesh,
        scratch_types=[
            pltpu.VMEM((2, CH, H), jnp.float32),
            pltpu.VMEM((2, CH, H), jnp.float32),
            pltpu.VMEM((2, CH // 2, L), jnp.float32),
            pltpu.VMEM((2, CH), jnp.int32),
            pltpu.VMEM((2, CH), jnp.int32),
            pltpu.VMEM((8, H), jnp.float32),
            pltpu.VMEM_SHARED((N_NODES, H), jnp.float32),
            pltpu.SemaphoreType.DMA,
            pltpu.SemaphoreType.DMA,
            pltpu.SemaphoreType.DMA,
            pltpu.SemaphoreType.DMA,
        ],
    )
    def body(x_hbm, rbf_hbm, idx_hbm, wt_hbm, out_hbm,
             xb, hb, rb, ib, ibs, wt, acc, si0, si1, ss0, ss1):
        cid = lax.axis_index("c")
        sid = lax.axis_index("s")
        wid = sid * NC + cid
        sin = (si0, si1)
        ssc = (ss0, ss1)

        pltpu.sync_copy(wt_hbm, wt)

        def zrow(j, _):
            for k in range(H // L):
                hb[0, j, pl.ds(k * L, L)] = jnp.zeros((L,), jnp.float32)
            return 0
        lax.fori_loop(0, CH, zrow, 0)

        @pl.when(sid < NS - 1)
        def _():
            for j in range(13):
                pltpu.sync_copy(hb.at[0, pl.ds(0, 48)],
                                acc.at[pl.ds(sid * 624 + j * 48, 48)])

        @pl.when(sid == NS - 1)
        def _():
            for j in range(10):
                pltpu.sync_copy(hb.at[0, pl.ds(0, 64)],
                                acc.at[pl.ds(9360 + j * 64, 64)])

        plsc.subcore_barrier()

        def chunk_id(c):
            return wid + c * NW

        def valid(c):
            return chunk_id(c) < nchunk

        def in_descs(c, b):
            t = chunk_id(c)
            return (
                pltpu.make_async_copy(x_hbm.at[pl.ds(t * CH, CH)],
                                      xb.at[b], sin[b]),
                pltpu.make_async_copy(
                    rbf_hbm.at[pl.ds(t * (CH // 2), CH // 2)],
                    rb.at[b], sin[b]),
                pltpu.make_async_copy(idx_hbm.at[pl.ds(t * CH, CH)],
                                      ib.at[b], sin[b]),
            )

        def sc_start(b):
            pltpu.async_copy(hb.at[b], acc.at[ibs.at[b]], ssc[b], add=True)

        def sc_wait(b):
            pltpu.make_async_copy(hb.at[b], acc.at[ibs.at[b]], ssc[b]).wait()

        def start_in(c, b):
            @pl.when(valid(c))
            def _():
                for d in in_descs(c, b):
                    d.start()

        def compute(b):
            def grp(g, _):
                rv0 = rb[b, 2 * g]
                rv1 = rb[b, 2 * g + 1]
                br = ([_bcast_lane(rv0, r) for r in range(6)]
                      + [_bcast_lane(rv0, 8 + r) for r in range(6)]
                      + [_bcast_lane(rv1, r) for r in range(6)]
                      + [_bcast_lane(rv1, 8 + r) for r in range(6)])
                e0 = 4 * g
                for k in range(H // L):
                    ds = pl.ds(k * L, L)
                    wcol = [wt[r, ds] for r in range(6)]
                    for e in range(4):
                        w = br[6 * e] * wcol[0]
                        for r in range(1, 6):
                            w = w + br[6 * e + r] * wcol[r]
                        hb[b, e0 + e, ds] = xb[b, e0 + e, ds] * w
                return 0
            lax.fori_loop(0, CH // 4, grp, 0)

        start_in(0, 0)
        start_in(1, 1)

        def pair_body(j, _):
            for b in range(2):
                c = 2 * j + b

                @pl.when(valid(c))
                def _():
                    for d in in_descs(c, b):
                        d.wait()

                    @pl.when(c >= 2)
                    def _():
                        sc_wait(b)

                    compute(b)
                    for k in range(CH // L):
                        ibs[b, pl.ds(k * L, L)] = ib[b, pl.ds(k * L, L)]
                    sc_start(b)
                    start_in(c + 2, b)
            return 0
        lax.fori_loop(0, (ncmax + 1) // 2, pair_body, 0)

        sc_wait(0)
        sc_wait(1)

        plsc.subcore_barrier()

        @pl.when(sid < NS - 1)
        def _():
            pltpu.sync_copy(acc.at[pl.ds(sid * 624, 624)],
                            out_hbm.at[cid, pl.ds(sid * 624, 624)])

        @pl.when(sid == NS - 1)
        def _():
            pltpu.sync_copy(acc.at[pl.ds(9360, 640)],
                            out_hbm.at[cid, pl.ds(9360, 640)])

    return body(x, rbf_p, idx, wt8)


def _mlp(partials, W0, b0, W1, b1, W2, b2, W_out):
    BN = 2000
    grid = (N_NODES // BN,)

    def body(p_ref, w0, b0r, w1, b1r, w2, b2r, wo, o_ref):
        dn = (((1,), (1,)), ((), ()))
        h = p_ref[0] + p_ref[1]
        h = jnp.maximum(
            lax.dot_general(h, w0[...], dn, preferred_element_type=jnp.float32)
            + b0r[...], 0.0)
        h = jnp.maximum(
            lax.dot_general(h, w1[...], dn, preferred_element_type=jnp.float32)
            + b1r[...], 0.0)
        h = jnp.maximum(
            lax.dot_general(h, w2[...], dn, preferred_element_type=jnp.float32)
            + b2r[...], 0.0)
        o_ref[...] = lax.dot_general(h, wo[...], dn,
                                     preferred_element_type=jnp.float32)

    wspec = pl.BlockSpec((H, H), lambda i: (0, 0))
    bspec = pl.BlockSpec((1, H), lambda i: (0, 0))
    return pl.pallas_call(
        body,
        grid=grid,
        in_specs=[
            pl.BlockSpec((NC, BN, H), lambda i: (0, i, 0)),
            wspec, bspec, wspec, bspec, wspec, bspec, wspec,
        ],
        out_specs=pl.BlockSpec((BN, H), lambda i: (i, 0)),
        out_shape=jax.ShapeDtypeStruct((N_NODES, H), jnp.float32),
    )(partials, W0, b0, W1, b1, W2, b2, W_out)


def kernel(x, rbf, i, num_nodes, W_rbf, W0, b0, W1, b1, W2, b2, W_out):
    E = x.shape[0]
    idx = jnp.minimum(i, N_NODES - 1).astype(jnp.int32)
    rbf8 = jnp.concatenate(
        [rbf.astype(jnp.float32), jnp.zeros((E, 2), jnp.float32)], axis=1)
    rbf_p = rbf8.reshape(E // 2, 16)
    wt8 = jnp.zeros((8, H), jnp.float32).at[:6, :].set(W_rbf.T)

    partials = _edge_scatter(x, rbf_p, idx, wt8)
    return _mlp(partials, W0, b0.reshape(1, H), W1, b1.reshape(1, H),
                W2, b2.reshape(1, H), W_out)
